# baseline (speedup 1.0000x reference)
import functools
import math

import jax
import jax.numpy as jnp
from jax import lax
from jax.experimental import pallas as pl
from jax.experimental.pallas import tpu as pltpu
from jax.experimental.pallas import tpu_sc as plsc


def _make_sc_gather(V, D, Vf, Dfp, N):
    info = plsc.get_sparse_core_info()
    NC, NS = info.num_cores, info.num_subcores
    NW = NC * NS
    assert N % NW == 0
    T = N // NW
    CH = 32
    NCHUNK = T // CH
    FCH = min(T, 128)
    NFCH = T // FCH

    mesh = plsc.VectorSubcoreMesh(core_axis_name="c", subcore_axis_name="s")

    @functools.partial(
        pl.kernel,
        mesh=mesh,
        out_type=[
            jax.ShapeDtypeStruct((N, D), jnp.float32),
            jax.ShapeDtypeStruct((N, Dfp), jnp.float32),
            jax.ShapeDtypeStruct((N, Dfp), jnp.float32),
        ],
        scratch_types=[
            pltpu.VMEM((T,), jnp.int32),
            pltpu.VMEM((T,), jnp.int32),
            pltpu.VMEM((T,), jnp.int32),
            pltpu.VMEM((CH, D), jnp.float32),
            pltpu.VMEM((CH, D), jnp.float32),
            pltpu.VMEM((FCH, Dfp), jnp.float32),
            pltpu.VMEM((FCH, Dfp), jnp.float32),
            pltpu.SemaphoreType.DMA,
            pltpu.SemaphoreType.DMA,
            pltpu.SemaphoreType.DMA,
            pltpu.SemaphoreType.DMA,
            pltpu.SemaphoreType.DMA,
            pltpu.SemaphoreType.DMA,
            pltpu.SemaphoreType.DMA,
        ],
    )
    def sc_gather(w_hbm, f1_hbm, f2_hbm, i0_hbm, i1_hbm, i2_hbm,
                  e0_hbm, e1_hbm, e2_hbm,
                  i0_v, i1_v, i2_v, wbuf0, wbuf1, fbuf1, fbuf2,
                  sem0, sem1, semw0, semw1, semf, semfw1, semfw2):
        wid = lax.axis_index("s") * NC + lax.axis_index("c")
        base = wid * T
        pltpu.sync_copy(i0_hbm.at[pl.ds(base, T)], i0_v)
        pltpu.sync_copy(i1_hbm.at[pl.ds(base, T)], i1_v)
        pltpu.sync_copy(i2_hbm.at[pl.ds(base, T)], i2_v)

        fcopies = []
        for j in range(NFCH):
            for idx_v, tbl, out, fbuf, wsem in (
                    (i1_v, f1_hbm, e1_hbm, fbuf1, semfw1),
                    (i2_v, f2_hbm, e2_hbm, fbuf2, semfw2)):
                pltpu.async_copy(
                    tbl.at[idx_v.at[pl.ds(j * FCH, FCH)]], fbuf, semf).wait()
                fcopies.append(pltpu.async_copy(
                    fbuf, out.at[pl.ds(base + j * FCH, FCH)], wsem))

        bufs = (wbuf0, wbuf1)
        gsems = (sem0, sem1)
        wsems = (semw0, semw1)

        def start_gather(cidx):
            b = cidx % 2
            return pltpu.async_copy(
                w_hbm.at[i0_v.at[pl.ds(cidx * CH, CH)]], bufs[b], gsems[b])

        gcp = [None, None]
        wcp = [None, None]
        gcp[0] = start_gather(0)
        for cidx in range(NCHUNK):
            b = cidx % 2
            gcp[b].wait()
            wcp[b] = pltpu.async_copy(
                bufs[b], e0_hbm.at[pl.ds(base + cidx * CH, CH)], wsems[b])
            nxt = cidx + 1
            if nxt < NCHUNK:
                nb = nxt % 2
                if wcp[nb] is not None:
                    wcp[nb].wait()
                gcp[nb] = start_gather(nxt)
        for cp in wcp:
            if cp is not None:
                cp.wait()
        for cp in fcopies:
            cp.wait()

    return sc_gather


def _tc_body(e0_ref, e1_ref, e2_ref, w0_ref, w1_ref, w2_ref, b_ref, pe_ref,
             o_ref, *, scale):
    e0 = e0_ref[...].astype(jnp.bfloat16)
    e1 = e1_ref[...].astype(jnp.bfloat16)
    e2 = e2_ref[...].astype(jnp.bfloat16)
    acc = jnp.dot(e0, w0_ref[...], preferred_element_type=jnp.float32)
    acc += jnp.dot(e1, w1_ref[...], preferred_element_type=jnp.float32)
    acc += jnp.dot(e2, w2_ref[...], preferred_element_type=jnp.float32)
    h = jnp.maximum(acc + b_ref[...], 0.0)
    o_ref[...] = h * scale + pe_ref[...]


def _tc_body_acc(e0_ref, e1_ref, e2_ref, w0_ref, w1_ref, w2_ref, b_ref,
                 pe_ref, buf_ref, o_ref, *, scale):
    del buf_ref
    _tc_body(e0_ref, e1_ref, e2_ref, w0_ref, w1_ref, w2_ref, b_ref, pe_ref,
             o_ref, scale=scale)


def _tc_mlp_chunk(e0, e1, e2, W0, W1, W2, b, peL, chunk, nchunks, buf):
    Nc, D = e0.shape
    Dfp = e1.shape[1]
    N = Nc * nchunks
    bm = 512
    nblk = Nc // bm
    scale = math.sqrt(D)
    in_specs = [
        pl.BlockSpec((bm, D), lambda i: (i, 0)),
        pl.BlockSpec((bm, Dfp), lambda i: (i, 0)),
        pl.BlockSpec((bm, Dfp), lambda i: (i, 0)),
        pl.BlockSpec((D, D), lambda i: (0, 0)),
        pl.BlockSpec((Dfp, D), lambda i: (0, 0)),
        pl.BlockSpec((Dfp, D), lambda i: (0, 0)),
        pl.BlockSpec((1, D), lambda i: (0, 0)),
        pl.BlockSpec((bm, D), lambda i: (i, 0)),
    ]
    args = [e0, e1, e2, W0, W1, W2, b, peL]
    if buf is None:
        body = functools.partial(_tc_body, scale=scale)
        aliases = {}
    else:
        body = functools.partial(_tc_body_acc, scale=scale)
        in_specs.append(pl.BlockSpec(memory_space=pl.ANY))
        args.append(buf)
        aliases = {8: 0}
    return pl.pallas_call(
        body,
        grid=(nblk,),
        in_specs=in_specs,
        out_specs=pl.BlockSpec((bm, D), lambda i, c=chunk: (c * nblk + i, 0)),
        out_shape=jax.ShapeDtypeStruct((N, D), jnp.float32),
        input_output_aliases=aliases,
        compiler_params=pltpu.CompilerParams(
            dimension_semantics=("arbitrary",),
        ),
    )(*args)


def kernel(src, W_word, W_f1, W_f2, W_mlp, b_mlp, pe):
    B, L, _ = src.shape
    N = B * L
    V, D = W_word.shape
    Vf, Df = W_f1.shape

    idx = src.reshape(N, 3)
    i0 = idx[:, 0]
    i1 = idx[:, 1]
    i2 = idx[:, 2]

    Dfp = 128
    W_f1p = jnp.pad(W_f1, ((0, 0), (0, Dfp - Df)))
    W_f2p = jnp.pad(W_f2, ((0, 0), (0, Dfp - Df)))

    W0 = W_mlp[:D].astype(jnp.bfloat16)
    W1 = jnp.pad(W_mlp[D:D + Df], ((0, Dfp - Df), (0, 0))).astype(jnp.bfloat16)
    W2 = jnp.pad(W_mlp[D + Df:], ((0, Dfp - Df), (0, 0))).astype(jnp.bfloat16)
    b = b_mlp.reshape(1, D)
    peL = pe[:L]

    nchunks = B
    Nc = N // nchunks
    sc_gather = _make_sc_gather(V, D, Vf, Dfp, Nc)

    gathered = []
    for c in range(nchunks):
        sl = pl.ds(c * Nc, Nc)
        gathered.append(
            sc_gather(W_word, W_f1p, W_f2p,
                      lax.dynamic_slice(i0, (c * Nc,), (Nc,)),
                      lax.dynamic_slice(i1, (c * Nc,), (Nc,)),
                      lax.dynamic_slice(i2, (c * Nc,), (Nc,))))

    buf = None
    for c in range(nchunks):
        e0, e1, e2 = gathered[c]
        buf = _tc_mlp_chunk(e0, e1, e2, W0, W1, W2, b, peL, c, nchunks, buf)

    return buf.reshape(B, L, D)


# speedup vs baseline: 1.0718x; 1.0718x over previous
import functools
import math

import jax
import jax.numpy as jnp
from jax import lax
from jax.experimental import pallas as pl
from jax.experimental.pallas import tpu as pltpu
from jax.experimental.pallas import tpu_sc as plsc


def _make_sc_gather(V, D, Vf, Dfp, N):
    info = plsc.get_sparse_core_info()
    NC, NS = info.num_cores, info.num_subcores
    NW = NC * NS
    assert N % NW == 0
    T = N // NW
    CH = 32
    NCHUNK = T // CH
    FCH = min(T, 128)
    NFCH = T // FCH

    mesh = plsc.VectorSubcoreMesh(core_axis_name="c", subcore_axis_name="s")

    @functools.partial(
        pl.kernel,
        mesh=mesh,
        out_type=[
            jax.ShapeDtypeStruct((N, D), jnp.float32),
            jax.ShapeDtypeStruct((N, Dfp), jnp.float32),
            jax.ShapeDtypeStruct((N, Dfp), jnp.float32),
        ],
        scratch_types=[
            pltpu.VMEM((T,), jnp.int32),
            pltpu.VMEM((T,), jnp.int32),
            pltpu.VMEM((T,), jnp.int32),
            pltpu.VMEM((CH, D), jnp.float32),
            pltpu.VMEM((CH, D), jnp.float32),
            pltpu.VMEM((FCH, Dfp), jnp.float32),
            pltpu.VMEM((FCH, Dfp), jnp.float32),
            pltpu.SemaphoreType.DMA,
            pltpu.SemaphoreType.DMA,
            pltpu.SemaphoreType.DMA,
            pltpu.SemaphoreType.DMA,
            pltpu.SemaphoreType.DMA,
            pltpu.SemaphoreType.DMA,
            pltpu.SemaphoreType.DMA,
        ],
    )
    def sc_gather(w_hbm, f1_hbm, f2_hbm, i0_hbm, i1_hbm, i2_hbm,
                  e0_hbm, e1_hbm, e2_hbm,
                  i0_v, i1_v, i2_v, wbuf0, wbuf1, fbuf1, fbuf2,
                  sem0, sem1, semw0, semw1, semf, semfw1, semfw2):
        wid = lax.axis_index("s") * NC + lax.axis_index("c")
        base = wid * T
        pltpu.sync_copy(i0_hbm.at[pl.ds(base, T)], i0_v)
        pltpu.sync_copy(i1_hbm.at[pl.ds(base, T)], i1_v)
        pltpu.sync_copy(i2_hbm.at[pl.ds(base, T)], i2_v)

        fcopies = []
        for j in range(NFCH):
            for idx_v, tbl, out, fbuf, wsem in (
                    (i1_v, f1_hbm, e1_hbm, fbuf1, semfw1),
                    (i2_v, f2_hbm, e2_hbm, fbuf2, semfw2)):
                pltpu.async_copy(
                    tbl.at[idx_v.at[pl.ds(j * FCH, FCH)]], fbuf, semf).wait()
                fcopies.append(pltpu.async_copy(
                    fbuf, out.at[pl.ds(base + j * FCH, FCH)], wsem))

        bufs = (wbuf0, wbuf1)
        gsems = (sem0, sem1)
        wsems = (semw0, semw1)

        def start_gather(cidx):
            b = cidx % 2
            return pltpu.async_copy(
                w_hbm.at[i0_v.at[pl.ds(cidx * CH, CH)]], bufs[b], gsems[b])

        gcp = [None, None]
        wcp = [None, None]
        gcp[0] = start_gather(0)
        for cidx in range(NCHUNK):
            b = cidx % 2
            gcp[b].wait()
            wcp[b] = pltpu.async_copy(
                bufs[b], e0_hbm.at[pl.ds(base + cidx * CH, CH)], wsems[b])
            nxt = cidx + 1
            if nxt < NCHUNK:
                nb = nxt % 2
                if wcp[nb] is not None:
                    wcp[nb].wait()
                gcp[nb] = start_gather(nxt)
        for cp in wcp:
            if cp is not None:
                cp.wait()
        for cp in fcopies:
            cp.wait()

    return sc_gather


def _tc_body(e0_ref, e1_ref, e2_ref, w0_ref, w1_ref, w2_ref, b_ref, pe_ref,
             o_ref, *, scale):
    acc = jnp.dot(e0_ref[...], w0_ref[...], preferred_element_type=jnp.float32)
    acc += jnp.dot(e1_ref[...], w1_ref[...], preferred_element_type=jnp.float32)
    acc += jnp.dot(e2_ref[...], w2_ref[...], preferred_element_type=jnp.float32)
    h = jnp.maximum(acc + b_ref[...], 0.0)
    o_ref[...] = h * scale + pe_ref[...]


def _tc_body_acc(e0_ref, e1_ref, e2_ref, w0_ref, w1_ref, w2_ref, b_ref,
                 pe_ref, buf_ref, o_ref, *, scale):
    del buf_ref
    _tc_body(e0_ref, e1_ref, e2_ref, w0_ref, w1_ref, w2_ref, b_ref, pe_ref,
             o_ref, scale=scale)


def _tc_mlp_chunk(e0, e1, e2, W_mlp, W1, W2, b, pe, chunk, nchunks, B, buf):
    Nc, D = e0.shape
    Dfp = e1.shape[1]
    N = Nc * nchunks
    bm = Nc // B
    nblk_out = N // bm
    scale = math.sqrt(D)
    in_specs = [
        pl.BlockSpec((bm, D), lambda i: (i, 0)),
        pl.BlockSpec((bm, Dfp), lambda i: (i, 0)),
        pl.BlockSpec((bm, Dfp), lambda i: (i, 0)),
        pl.BlockSpec((D, D), lambda i: (0, 0)),
        pl.BlockSpec((Dfp, D), lambda i: (0, 0)),
        pl.BlockSpec((Dfp, D), lambda i: (0, 0)),
        pl.BlockSpec((1, D), lambda i: (0, 0)),
        pl.BlockSpec((bm, D), lambda i, c=chunk: (c, 0)),
    ]
    args = [e0, e1, e2, W_mlp, W1, W2, b, pe]
    nper = nchunks
    out_spec = pl.BlockSpec(
        (bm, D), lambda i, c=chunk, k=nper: (i * k + c, 0))
    if buf is None:
        body = functools.partial(_tc_body, scale=scale)
        aliases = {}
    else:
        body = functools.partial(_tc_body_acc, scale=scale)
        in_specs.append(pl.BlockSpec(memory_space=pl.ANY))
        args.append(buf)
        aliases = {8: 0}
    return pl.pallas_call(
        body,
        grid=(B,),
        in_specs=in_specs,
        out_specs=out_spec,
        out_shape=jax.ShapeDtypeStruct((N, D), jnp.float32),
        input_output_aliases=aliases,
        compiler_params=pltpu.CompilerParams(
            dimension_semantics=("arbitrary",),
        ),
    )(*args)


def kernel(src, W_word, W_f1, W_f2, W_mlp, b_mlp, pe):
    B, L, _ = src.shape
    N = B * L
    V, D = W_word.shape
    Vf, Df = W_f1.shape

    nchunks = 4
    P = L // nchunks
    Nc = B * P

    idx_t = src.reshape(B, nchunks, P, 3).transpose(1, 3, 0, 2).reshape(
        nchunks, 3, Nc)

    Dfp = 128
    W_f1p = jnp.pad(W_f1, ((0, 0), (0, Dfp - Df)))
    W_f2p = jnp.pad(W_f2, ((0, 0), (0, Dfp - Df)))
    W1 = jnp.pad(W_mlp[D:D + Df], ((0, Dfp - Df), (0, 0)))
    W2 = jnp.pad(W_mlp[D + Df:], ((0, Dfp - Df), (0, 0)))
    b = b_mlp.reshape(1, D)

    sc_gather = _make_sc_gather(V, D, Vf, Dfp, Nc)

    gathered = []
    for c in range(nchunks):
        gathered.append(
            sc_gather(W_word, W_f1p, W_f2p,
                      idx_t[c, 0], idx_t[c, 1], idx_t[c, 2]))

    buf = None
    for c in range(nchunks):
        e0, e1, e2 = gathered[c]
        buf = _tc_mlp_chunk(e0, e1, e2, W_mlp, W1, W2, b, pe, c, nchunks,
                            B, buf)

    return buf.reshape(B, L, D)


# speedup vs baseline: 1.1851x; 1.1057x over previous
import functools
import math

import jax
import jax.numpy as jnp
from jax import lax
from jax.experimental import pallas as pl
from jax.experimental.pallas import tpu as pltpu
from jax.experimental.pallas import tpu_sc as plsc

_NCHUNKS = 2


def _make_sc_gather(V, D, Vf, Dfp, N):
    info = plsc.get_sparse_core_info()
    NC, NS = info.num_cores, info.num_subcores
    NW = NC * NS
    assert N % NW == 0
    T = N // NW
    CH = 32
    NCHUNK = T // CH
    FCH = min(T, 128)
    NFCH = T // FCH

    mesh = plsc.VectorSubcoreMesh(core_axis_name="c", subcore_axis_name="s")

    @functools.partial(
        pl.kernel,
        mesh=mesh,
        out_type=[
            jax.ShapeDtypeStruct((N, D), jnp.float32),
            jax.ShapeDtypeStruct((N, Dfp), jnp.float32),
            jax.ShapeDtypeStruct((N, Dfp), jnp.float32),
        ],
        scratch_types=[
            pltpu.VMEM((3, T), jnp.int32),
            pltpu.VMEM((CH, D), jnp.float32),
            pltpu.VMEM((CH, D), jnp.float32),
            pltpu.VMEM((FCH, Dfp), jnp.float32),
            pltpu.VMEM((FCH, Dfp), jnp.float32),
            pltpu.SemaphoreType.DMA,
            pltpu.SemaphoreType.DMA,
            pltpu.SemaphoreType.DMA,
            pltpu.SemaphoreType.DMA,
            pltpu.SemaphoreType.DMA,
            pltpu.SemaphoreType.DMA,
            pltpu.SemaphoreType.DMA,
        ],
    )
    def sc_gather(w_hbm, f1_hbm, f2_hbm, idx_hbm,
                  e0_hbm, e1_hbm, e2_hbm,
                  idx_v, wbuf0, wbuf1, fbuf1, fbuf2,
                  sem0, sem1, semw0, semw1, semf, semfw1, semfw2):
        wid = lax.axis_index("s") * NC + lax.axis_index("c")
        base = wid * T
        pltpu.sync_copy(idx_hbm.at[:, pl.ds(base, T)], idx_v)

        fcopies = []
        for j in range(NFCH):
            for row, tbl, out, fbuf, wsem in (
                    (1, f1_hbm, e1_hbm, fbuf1, semfw1),
                    (2, f2_hbm, e2_hbm, fbuf2, semfw2)):
                pltpu.async_copy(
                    tbl.at[idx_v.at[row, pl.ds(j * FCH, FCH)]], fbuf,
                    semf).wait()
                fcopies.append(pltpu.async_copy(
                    fbuf, out.at[pl.ds(base + j * FCH, FCH)], wsem))

        bufs = (wbuf0, wbuf1)
        gsems = (sem0, sem1)
        wsems = (semw0, semw1)

        def start_gather(cidx):
            b = cidx % 2
            return pltpu.async_copy(
                w_hbm.at[idx_v.at[0, pl.ds(cidx * CH, CH)]], bufs[b],
                gsems[b])

        gcp = [None, None]
        wcp = [None, None]
        gcp[0] = start_gather(0)
        for cidx in range(NCHUNK):
            b = cidx % 2
            gcp[b].wait()
            wcp[b] = pltpu.async_copy(
                bufs[b], e0_hbm.at[pl.ds(base + cidx * CH, CH)], wsems[b])
            nxt = cidx + 1
            if nxt < NCHUNK:
                nb = nxt % 2
                if wcp[nb] is not None:
                    wcp[nb].wait()
                gcp[nb] = start_gather(nxt)
        for cp in wcp:
            if cp is not None:
                cp.wait()
        for cp in fcopies:
            cp.wait()

    return sc_gather


def _tc_body(e0_ref, e1_ref, e2_ref, w0_ref, w1_ref, w2_ref, b_ref, pe_ref,
             o_ref, *, scale):
    acc = jnp.dot(e0_ref[...], w0_ref[...], preferred_element_type=jnp.float32)
    acc += jnp.dot(e1_ref[...], w1_ref[...], preferred_element_type=jnp.float32)
    acc += jnp.dot(e2_ref[...], w2_ref[...], preferred_element_type=jnp.float32)
    h = jnp.maximum(acc + b_ref[...], 0.0)
    o_ref[...] = h * scale + pe_ref[...]


def _tc_body_acc(e0_ref, e1_ref, e2_ref, w0_ref, w1_ref, w2_ref, b_ref,
                 pe_ref, buf_ref, o_ref, *, scale):
    del buf_ref
    _tc_body(e0_ref, e1_ref, e2_ref, w0_ref, w1_ref, w2_ref, b_ref, pe_ref,
             o_ref, scale=scale)


def _tc_mlp_chunk(e0, e1, e2, W_mlp, W1, W2, b, pe, chunk, nchunks, B, buf):
    Nc, D = e0.shape
    Dfp = e1.shape[1]
    N = Nc * nchunks
    bm = Nc // B
    scale = math.sqrt(D)
    in_specs = [
        pl.BlockSpec((bm, D), lambda i: (i, 0)),
        pl.BlockSpec((bm, Dfp), lambda i: (i, 0)),
        pl.BlockSpec((bm, Dfp), lambda i: (i, 0)),
        pl.BlockSpec((D, D), lambda i: (0, 0)),
        pl.BlockSpec((Dfp, D), lambda i: (0, 0)),
        pl.BlockSpec((Dfp, D), lambda i: (0, 0)),
        pl.BlockSpec((1, D), lambda i: (0, 0)),
        pl.BlockSpec((bm, D), lambda i, c=chunk: (c, 0)),
    ]
    args = [e0, e1, e2, W_mlp, W1, W2, b, pe]
    out_spec = pl.BlockSpec(
        (bm, D), lambda i, c=chunk, k=nchunks: (i * k + c, 0))
    if buf is None:
        body = functools.partial(_tc_body, scale=scale)
        aliases = {}
    else:
        body = functools.partial(_tc_body_acc, scale=scale)
        in_specs.append(pl.BlockSpec(memory_space=pl.ANY))
        args.append(buf)
        aliases = {8: 0}
    return pl.pallas_call(
        body,
        grid=(B,),
        in_specs=in_specs,
        out_specs=out_spec,
        out_shape=jax.ShapeDtypeStruct((N, D), jnp.float32),
        input_output_aliases=aliases,
        compiler_params=pltpu.CompilerParams(
            dimension_semantics=("arbitrary",),
        ),
    )(*args)


def kernel(src, W_word, W_f1, W_f2, W_mlp, b_mlp, pe):
    B, L, _ = src.shape
    N = B * L
    V, D = W_word.shape
    Vf, Df = W_f1.shape

    nchunks = _NCHUNKS
    P = L // nchunks
    Nc = B * P

    idx_t = src.reshape(B, nchunks, P, 3).transpose(1, 3, 0, 2).reshape(
        nchunks, 3, Nc)

    Dfp = 128
    W_f1p = jnp.pad(W_f1, ((0, 0), (0, Dfp - Df)))
    W_f2p = jnp.pad(W_f2, ((0, 0), (0, Dfp - Df)))
    W1 = jnp.pad(W_mlp[D:D + Df], ((0, Dfp - Df), (0, 0)))
    W2 = jnp.pad(W_mlp[D + Df:], ((0, Dfp - Df), (0, 0)))
    b = b_mlp.reshape(1, D)

    sc_gather = _make_sc_gather(V, D, Vf, Dfp, Nc)

    gathered = []
    for c in range(nchunks):
        gathered.append(sc_gather(W_word, W_f1p, W_f2p, idx_t[c]))

    buf = None
    for c in range(nchunks):
        e0, e1, e2 = gathered[c]
        buf = _tc_mlp_chunk(e0, e1, e2, W_mlp, W1, W2, b, pe, c, nchunks,
                            B, buf)

    return buf.reshape(B, L, D)
